# initial kernel scaffold (unmeasured)
import jax
import jax.numpy as jnp
from jax import lax
from jax.experimental import pallas as pl
from jax.experimental.pallas import tpu as pltpu

N_DEV = 8


def kernel(x, w_mat, scale_x, scale_w):
    m_total, k_per = x.shape
    _, n = w_mat.shape
    m_per = m_total // N_DEV

    def body(x_ref, w_ref, sx_ref, sw_ref, out_ref,
             send_buf, comm_buf, send_sems, recv_sems, credit_sem):
        my = lax.axis_index("i")
        left = (my - 1) % N_DEV
        right = (my + 1) % N_DEV

        barrier = pltpu.get_barrier_semaphore()
        for nbr in (left, right):
            pl.semaphore_signal(barrier, inc=1, device_id=(nbr,),
                                device_id_type=pl.DeviceIdType.MESH)
        pl.semaphore_wait(barrier, 2)

        def gemm(c):
            xa = pl.load(x_ref, (pl.ds(c * m_per, m_per), slice(None)))
            return lax.dot_general(xa, w_ref[...], (((1,), (0,)), ((), ())),
                                   preferred_element_type=jnp.float32)

        def rdma(slot, target):
            return pltpu.make_async_remote_copy(
                src_ref=send_buf.at[slot],
                dst_ref=comm_buf.at[slot],
                send_sem=send_sems.at[slot],
                recv_sem=recv_sems.at[slot],
                device_id=(target,),
                device_id_type=pl.DeviceIdType.MESH,
            )

        c0 = (my - 1) % N_DEV
        send_buf[0] = gemm(c0).astype(jnp.bfloat16)
        rdma(0, right).start()

        for s in range(1, N_DEV):
            c = (my - 1 - s) % N_DEV
            part = gemm(c)
            rslot = (s - 1) % 2
            rdma(rslot, left).wait_recv()
            acc = comm_buf[rslot].astype(jnp.float32) + part
            if s < N_DEV - 1:
                sslot = s % 2
                if s >= 2:
                    pl.semaphore_wait(credit_sem, 1)
                    rdma(sslot, right).wait_send()
                send_buf[sslot] = acc.astype(jnp.bfloat16)
                if s <= N_DEV - 3:
                    pl.semaphore_signal(credit_sem, inc=1, device_id=(left,),
                                        device_id_type=pl.DeviceIdType.MESH)
                rdma(sslot, right).start()
            else:
                y = acc * (sx_ref[0] * sw_ref[0])
                out_ref[...] = y / (1.0 + jnp.exp(-jnp.clip(y, -60.0, 60.0)))

        rdma(1, right).wait_send()
        rdma(0, right).wait_send()

    return pl.pallas_call(
        body,
        out_shape=jax.ShapeDtypeStruct((m_per, n), jnp.float32),
        in_specs=[
            pl.BlockSpec(memory_space=pltpu.VMEM),
            pl.BlockSpec(memory_space=pltpu.VMEM),
            pl.BlockSpec(memory_space=pltpu.SMEM),
            pl.BlockSpec(memory_space=pltpu.SMEM),
        ],
        out_specs=pl.BlockSpec(memory_space=pltpu.VMEM),
        scratch_shapes=[
            pltpu.VMEM((2, m_per, n), jnp.bfloat16),
            pltpu.VMEM((2, m_per, n), jnp.bfloat16),
            pltpu.SemaphoreType.DMA((2,)),
            pltpu.SemaphoreType.DMA((2,)),
            pltpu.SemaphoreType.REGULAR,
        ],
        compiler_params=pltpu.CompilerParams(collective_id=0),
    )(x, w_mat, scale_x, scale_w)


# baseline (device time: 418886 ns/iter reference)
import jax
import jax.numpy as jnp
from jax import lax
from jax.experimental import pallas as pl
from jax.experimental.pallas import tpu as pltpu

N_DEV = 8
N_BLOCKS = 8
RINGS_PER_DIR = 4
STEPS = N_DEV * RINGS_PER_DIR
N_HOPS = RINGS_PER_DIR * (N_DEV - 1)


def kernel(x, w_mat, scale_x, scale_w):
    m_total, k_per = x.shape
    _, n = w_mat.shape
    m_per = m_total // N_DEV
    n_blk = n // N_BLOCKS

    x8 = x.astype(jnp.float8_e5m2)
    w8 = w_mat.astype(jnp.float8_e5m2)

    def body(x_ref, w_ref, sx_ref, sw_ref, out_ref,
             send_cw, comm_cw, ssem_cw, rsem_cw, credit_cw,
             send_ccw, comm_ccw, ssem_ccw, rsem_ccw, credit_ccw):
        my = lax.axis_index("i")
        left = (my - 1) % N_DEV
        right = (my + 1) % N_DEV

        barrier = pltpu.get_barrier_semaphore()
        for nbr in (left, right):
            pl.semaphore_signal(barrier, inc=1, device_id=(nbr,),
                                device_id_type=pl.DeviceIdType.MESH)
        pl.semaphore_wait(barrier, 2)

        scale = sx_ref[0] * sw_ref[0]

        def gemm(c, blk):
            xa = x_ref[pl.ds(c * m_per, m_per), :]
            wb = w_ref[:, blk * n_blk:(blk + 1) * n_blk]
            return lax.dot_general(xa, wb, (((1,), (0,)), ((), ())),
                                   preferred_element_type=jnp.float32)

        dirs = [
            dict(send_buf=send_cw, comm_buf=comm_cw, ssem=ssem_cw,
                 rsem=rsem_cw, credit=credit_cw, to=right, frm=left,
                 blk0=0, sign=1),
            dict(send_buf=send_ccw, comm_buf=comm_ccw, ssem=ssem_ccw,
                 rsem=rsem_ccw, credit=credit_ccw, to=left, frm=right,
                 blk0=RINGS_PER_DIR, sign=-1),
        ]

        def rdma(d, slot, target):
            return pltpu.make_async_remote_copy(
                src_ref=d["send_buf"].at[slot],
                dst_ref=d["comm_buf"].at[slot],
                send_sem=d["ssem"].at[slot],
                recv_sem=d["rsem"].at[slot],
                device_id=(target,),
                device_id_type=pl.DeviceIdType.MESH,
            )

        for t in range(STEPS):
            s = t % N_DEV
            r = t // N_DEV
            h_send = t - r
            h_recv = h_send - 1
            for d in dirs:
                c = (my - d["sign"] * (1 + s)) % N_DEV
                blk = d["blk0"] + r
                part = gemm(c, blk)
                if s == 0:
                    acc = part
                else:
                    rslot = h_recv % 2
                    rdma(d, rslot, d["frm"]).wait_recv()
                    acc = d["comm_buf"][rslot].astype(jnp.float32) + part
                if s < N_DEV - 1:
                    sslot = h_send % 2
                    if h_send >= 2:
                        pl.semaphore_wait(d["credit"], 1)
                        rdma(d, sslot, d["to"]).wait_send()
                    d["send_buf"][sslot] = acc.astype(jnp.bfloat16)
                    if s > 0 and h_recv <= N_HOPS - 3:
                        pl.semaphore_signal(
                            d["credit"], inc=1, device_id=(d["frm"],),
                            device_id_type=pl.DeviceIdType.MESH)
                    rdma(d, sslot, d["to"]).start()
                else:
                    y = acc * scale
                    out_ref[:, blk * n_blk:(blk + 1) * n_blk] = (
                        y / (1.0 + jnp.exp(-jnp.clip(y, -60.0, 60.0))))
                    if h_recv <= N_HOPS - 3:
                        pl.semaphore_signal(
                            d["credit"], inc=1, device_id=(d["frm"],),
                            device_id_type=pl.DeviceIdType.MESH)

        for d in dirs:
            rdma(d, (N_HOPS - 2) % 2, d["to"]).wait_send()
            rdma(d, (N_HOPS - 1) % 2, d["to"]).wait_send()

    comm_shape = (2, m_per, n_blk)
    return pl.pallas_call(
        body,
        out_shape=jax.ShapeDtypeStruct((m_per, n), jnp.float32),
        in_specs=[
            pl.BlockSpec(memory_space=pltpu.VMEM),
            pl.BlockSpec(memory_space=pltpu.VMEM),
            pl.BlockSpec(memory_space=pltpu.SMEM),
            pl.BlockSpec(memory_space=pltpu.SMEM),
        ],
        out_specs=pl.BlockSpec(memory_space=pltpu.VMEM),
        scratch_shapes=[
            pltpu.VMEM(comm_shape, jnp.bfloat16),
            pltpu.VMEM(comm_shape, jnp.bfloat16),
            pltpu.SemaphoreType.DMA((2,)),
            pltpu.SemaphoreType.DMA((2,)),
            pltpu.SemaphoreType.REGULAR,
            pltpu.VMEM(comm_shape, jnp.bfloat16),
            pltpu.VMEM(comm_shape, jnp.bfloat16),
            pltpu.SemaphoreType.DMA((2,)),
            pltpu.SemaphoreType.DMA((2,)),
            pltpu.SemaphoreType.REGULAR,
        ],
        compiler_params=pltpu.CompilerParams(
            collective_id=0, vmem_limit_bytes=40 * 1024 * 1024),
    )(x8, w8, scale_x, scale_w)


# device time: 357471 ns/iter; 1.1718x vs baseline; 1.1718x over previous
import jax
import jax.numpy as jnp
from jax import lax
from jax.experimental import pallas as pl
from jax.experimental.pallas import tpu as pltpu

N_DEV = 8
N_BLOCKS = 4
RINGS_PER_DIR = 2
SUB = 2
STEPS = N_DEV * RINGS_PER_DIR
N_HOPS = RINGS_PER_DIR * (N_DEV - 1)


def kernel(x, w_mat, scale_x, scale_w):
    m_total, k_per = x.shape
    _, n = w_mat.shape
    m_per = m_total // N_DEV
    n_blk = n // N_BLOCKS
    n_sub = n_blk // SUB

    x8 = x.astype(jnp.float8_e5m2)
    w8 = w_mat.astype(jnp.float8_e5m2)

    def body(x_ref, w_ref, sx_ref, sw_ref, out_ref,
             send_cw, comm_cw, ssem_cw, rsem_cw, credit_cw,
             send_ccw, comm_ccw, ssem_ccw, rsem_ccw, credit_ccw):
        my = lax.axis_index("i")
        left = (my - 1) % N_DEV
        right = (my + 1) % N_DEV

        barrier = pltpu.get_barrier_semaphore()
        for nbr in (left, right):
            pl.semaphore_signal(barrier, inc=1, device_id=(nbr,),
                                device_id_type=pl.DeviceIdType.MESH)
        pl.semaphore_wait(barrier, 2)

        scale = sx_ref[0] * sw_ref[0]

        def gemm(c, blk, p):
            xa = x_ref[pl.ds(c * m_per, m_per), :]
            lo = blk * n_blk + p * n_sub
            wb = w_ref[:, lo:lo + n_sub]
            return lax.dot_general(xa, wb, (((1,), (0,)), ((), ())),
                                   preferred_element_type=jnp.float32)

        dirs = [
            dict(send_buf=send_cw, comm_buf=comm_cw, ssem=ssem_cw,
                 rsem=rsem_cw, credit=credit_cw, to=right, frm=left,
                 blk0=0, sign=1),
            dict(send_buf=send_ccw, comm_buf=comm_ccw, ssem=ssem_ccw,
                 rsem=rsem_ccw, credit=credit_ccw, to=left, frm=right,
                 blk0=RINGS_PER_DIR, sign=-1),
        ]

        def rdma(d, slot, p, target):
            return pltpu.make_async_remote_copy(
                src_ref=d["send_buf"].at[slot, p],
                dst_ref=d["comm_buf"].at[slot, p],
                send_sem=d["ssem"].at[slot, p],
                recv_sem=d["rsem"].at[slot, p],
                device_id=(target,),
                device_id_type=pl.DeviceIdType.MESH,
            )

        for t in range(STEPS):
            s = t % N_DEV
            r = t // N_DEV
            h_send = t - r
            h_recv = h_send - 1
            for p in range(SUB):
                for d in dirs:
                    c = (my - d["sign"] * (1 + s)) % N_DEV
                    blk = d["blk0"] + r
                    part = gemm(c, blk, p)
                    if s == 0:
                        acc = part
                    else:
                        rslot = h_recv % 2
                        rdma(d, rslot, p, d["frm"]).wait_recv()
                        acc = (d["comm_buf"][rslot, p].astype(jnp.float32)
                               + part)
                    if s < N_DEV - 1:
                        sslot = h_send % 2
                        if h_send >= 2:
                            pl.semaphore_wait(d["credit"], 1)
                            rdma(d, sslot, p, d["to"]).wait_send()
                        d["send_buf"][sslot, p] = acc.astype(jnp.bfloat16)
                        if s > 0 and h_recv <= N_HOPS - 3:
                            pl.semaphore_signal(
                                d["credit"], inc=1, device_id=(d["frm"],),
                                device_id_type=pl.DeviceIdType.MESH)
                        rdma(d, sslot, p, d["to"]).start()
                    else:
                        y = acc * scale
                        lo = blk * n_blk + p * n_sub
                        out_ref[:, lo:lo + n_sub] = (
                            y / (1.0 + jnp.exp(-jnp.clip(y, -60.0, 60.0))))
                        if h_recv <= N_HOPS - 3:
                            pl.semaphore_signal(
                                d["credit"], inc=1, device_id=(d["frm"],),
                                device_id_type=pl.DeviceIdType.MESH)

        for d in dirs:
            for h in (N_HOPS - 2, N_HOPS - 1):
                for p in range(SUB):
                    rdma(d, h % 2, p, d["to"]).wait_send()

    comm_shape = (2, SUB, m_per, n_sub)
    return pl.pallas_call(
        body,
        out_shape=jax.ShapeDtypeStruct((m_per, n), jnp.float32),
        in_specs=[
            pl.BlockSpec(memory_space=pltpu.VMEM),
            pl.BlockSpec(memory_space=pltpu.VMEM),
            pl.BlockSpec(memory_space=pltpu.SMEM),
            pl.BlockSpec(memory_space=pltpu.SMEM),
        ],
        out_specs=pl.BlockSpec(memory_space=pltpu.VMEM),
        scratch_shapes=[
            pltpu.VMEM(comm_shape, jnp.bfloat16),
            pltpu.VMEM(comm_shape, jnp.bfloat16),
            pltpu.SemaphoreType.DMA((2, SUB)),
            pltpu.SemaphoreType.DMA((2, SUB)),
            pltpu.SemaphoreType.REGULAR,
            pltpu.VMEM(comm_shape, jnp.bfloat16),
            pltpu.VMEM(comm_shape, jnp.bfloat16),
            pltpu.SemaphoreType.DMA((2, SUB)),
            pltpu.SemaphoreType.DMA((2, SUB)),
            pltpu.SemaphoreType.REGULAR,
        ],
        compiler_params=pltpu.CompilerParams(
            collective_id=0, vmem_limit_bytes=40 * 1024 * 1024),
    )(x8, w8, scale_x, scale_w)


# device time: 353501 ns/iter; 1.1850x vs baseline; 1.0112x over previous
import jax
import jax.numpy as jnp
from jax import lax
from jax.experimental import pallas as pl
from jax.experimental.pallas import tpu as pltpu

N_DEV = 8
N_BLOCKS = 4
SUB = 2
N_HOPS = 2 * (N_DEV - 1)
BOOT = N_DEV - 1


def kernel(x, w_mat, scale_x, scale_w):
    m_total, k_per = x.shape
    _, n = w_mat.shape
    m_per = m_total // N_DEV
    n_blk = n // N_BLOCKS
    n_sub = n_blk // SUB

    x8 = x.astype(jnp.float8_e5m2)
    w8 = w_mat.astype(jnp.float8_e5m2)

    def body(x_ref, w_ref, sx_ref, sw_ref, out_ref,
             send_cw, comm_cw, ssem_cw, rsem_cw, credit_cw,
             send_ccw, comm_ccw, ssem_ccw, rsem_ccw, credit_ccw):
        my = lax.axis_index("i")
        left = (my - 1) % N_DEV
        right = (my + 1) % N_DEV

        barrier = pltpu.get_barrier_semaphore()
        for nbr in (left, right):
            pl.semaphore_signal(barrier, inc=1, device_id=(nbr,),
                                device_id_type=pl.DeviceIdType.MESH)
        pl.semaphore_wait(barrier, 2)

        scale = sx_ref[0] * sw_ref[0]

        def gemm(c, blk, p):
            xa = x_ref[pl.ds(c * m_per, m_per), :]
            lo = blk * n_blk + p * n_sub
            wb = w_ref[:, lo:lo + n_sub]
            return lax.dot_general(xa, wb, (((1,), (0,)), ((), ())),
                                   preferred_element_type=jnp.float32)

        dirs = [
            dict(send_buf=send_cw, comm_buf=comm_cw, ssem=ssem_cw,
                 rsem=rsem_cw, credit=credit_cw, to=right, frm=left,
                 blk0=0, sign=1),
            dict(send_buf=send_ccw, comm_buf=comm_ccw, ssem=ssem_ccw,
                 rsem=rsem_ccw, credit=credit_ccw, to=left, frm=right,
                 blk0=2, sign=-1),
        ]

        def rdma(d, h, p, target):
            sl = h % 2
            return pltpu.make_async_remote_copy(
                src_ref=d["send_buf"].at[sl, p],
                dst_ref=d["comm_buf"].at[sl, p],
                send_sem=d["ssem"].at[sl, p],
                recv_sem=d["rsem"].at[sl, p],
                device_id=(target,),
                device_id_type=pl.DeviceIdType.MESH,
            )

        def do_send(d, h, p, value):
            if h >= 2:
                pl.semaphore_wait(d["credit"], 1)
                rdma(d, h, p, d["to"]).wait_send()
            d["send_buf"][h % 2, p] = value.astype(jnp.bfloat16)
            rdma(d, h, p, d["to"]).start()

        def consume(d, h_recv, p, part):
            rdma(d, h_recv, p, d["frm"]).wait_recv()
            return d["comm_buf"][h_recv % 2, p].astype(jnp.float32) + part

        def credit_back(d, h_recv):
            if h_recv <= N_HOPS - 3:
                pl.semaphore_signal(d["credit"], inc=1,
                                    device_id=(d["frm"],),
                                    device_id_type=pl.DeviceIdType.MESH)

        def epilogue(d, h_recv, blk, p):
            part = gemm(my, blk, p)
            acc = consume(d, h_recv, p, part)
            y = acc * scale
            lo = blk * n_blk + p * n_sub
            out_ref[:, lo:lo + n_sub] = (
                y / (1.0 + jnp.exp(-jnp.clip(y, -60.0, 60.0))))
            credit_back(d, h_recv)

        for t in range(7):
            for p in range(SUB):
                for d in dirs:
                    c = (my - d["sign"] * (1 + t)) % N_DEV
                    part = gemm(c, d["blk0"], p)
                    acc = part if t == 0 else consume(d, t - 1, p, part)
                    do_send(d, t, p, acc)
                    if t > 0:
                        credit_back(d, t - 1)

        for p in range(SUB):
            for d in dirs:
                c = (my - d["sign"]) % N_DEV
                do_send(d, BOOT, p, gemm(c, d["blk0"] + 1, p))
        for p in range(SUB):
            for d in dirs:
                epilogue(d, BOOT - 1, d["blk0"], p)

        for t in range(8, 14):
            s = t - 7
            for p in range(SUB):
                for d in dirs:
                    c = (my - d["sign"] * (1 + s)) % N_DEV
                    part = gemm(c, d["blk0"] + 1, p)
                    acc = consume(d, t - 1, p, part)
                    do_send(d, t, p, acc)
                    credit_back(d, t - 1)

        for p in range(SUB):
            for d in dirs:
                epilogue(d, N_HOPS - 1, d["blk0"] + 1, p)

        for d in dirs:
            for h in (N_HOPS - 2, N_HOPS - 1):
                for p in range(SUB):
                    rdma(d, h, p, d["to"]).wait_send()

    comm_shape = (2, SUB, m_per, n_sub)
    return pl.pallas_call(
        body,
        out_shape=jax.ShapeDtypeStruct((m_per, n), jnp.float32),
        in_specs=[
            pl.BlockSpec(memory_space=pltpu.VMEM),
            pl.BlockSpec(memory_space=pltpu.VMEM),
            pl.BlockSpec(memory_space=pltpu.SMEM),
            pl.BlockSpec(memory_space=pltpu.SMEM),
        ],
        out_specs=pl.BlockSpec(memory_space=pltpu.VMEM),
        scratch_shapes=[
            pltpu.VMEM(comm_shape, jnp.bfloat16),
            pltpu.VMEM(comm_shape, jnp.bfloat16),
            pltpu.SemaphoreType.DMA((2, SUB)),
            pltpu.SemaphoreType.DMA((2, SUB)),
            pltpu.SemaphoreType.REGULAR,
            pltpu.VMEM(comm_shape, jnp.bfloat16),
            pltpu.VMEM(comm_shape, jnp.bfloat16),
            pltpu.SemaphoreType.DMA((2, SUB)),
            pltpu.SemaphoreType.DMA((2, SUB)),
            pltpu.SemaphoreType.REGULAR,
        ],
        compiler_params=pltpu.CompilerParams(
            collective_id=0, vmem_limit_bytes=40 * 1024 * 1024),
    )(x8, w8, scale_x, scale_w)


# device time: 352356 ns/iter; 1.1888x vs baseline; 1.0032x over previous
import jax
import jax.numpy as jnp
from jax import lax
from jax.experimental import pallas as pl
from jax.experimental.pallas import tpu as pltpu

N_DEV = 8
N_BLOCKS = 4
SUB = 4
N_HOPS = 2 * (N_DEV - 1)
BOOT = N_DEV - 1


def kernel(x, w_mat, scale_x, scale_w):
    m_total, k_per = x.shape
    _, n = w_mat.shape
    m_per = m_total // N_DEV
    n_blk = n // N_BLOCKS
    n_sub = n_blk // SUB

    x8 = x.astype(jnp.float8_e5m2)
    w8 = w_mat.astype(jnp.float8_e5m2)

    def body(x_ref, w_ref, sx_ref, sw_ref, out_ref,
             send_cw, comm_cw, ssem_cw, rsem_cw, credit_cw,
             send_ccw, comm_ccw, ssem_ccw, rsem_ccw, credit_ccw):
        my = lax.axis_index("i")
        left = (my - 1) % N_DEV
        right = (my + 1) % N_DEV

        barrier = pltpu.get_barrier_semaphore()
        for nbr in (left, right):
            pl.semaphore_signal(barrier, inc=1, device_id=(nbr,),
                                device_id_type=pl.DeviceIdType.MESH)
        pl.semaphore_wait(barrier, 2)

        scale = sx_ref[0] * sw_ref[0]

        def gemm(c, blk, p):
            xa = x_ref[pl.ds(c * m_per, m_per), :]
            lo = blk * n_blk + p * n_sub
            wb = w_ref[:, lo:lo + n_sub]
            return lax.dot_general(xa, wb, (((1,), (0,)), ((), ())),
                                   preferred_element_type=jnp.float32)

        dirs = [
            dict(send_buf=send_cw, comm_buf=comm_cw, ssem=ssem_cw,
                 rsem=rsem_cw, credit=credit_cw, to=right, frm=left,
                 blk0=0, sign=1),
            dict(send_buf=send_ccw, comm_buf=comm_ccw, ssem=ssem_ccw,
                 rsem=rsem_ccw, credit=credit_ccw, to=left, frm=right,
                 blk0=2, sign=-1),
        ]

        def rdma(d, h, p, target):
            sl = h % 2
            return pltpu.make_async_remote_copy(
                src_ref=d["send_buf"].at[sl, p],
                dst_ref=d["comm_buf"].at[sl, p],
                send_sem=d["ssem"].at[sl, p],
                recv_sem=d["rsem"].at[sl, p],
                device_id=(target,),
                device_id_type=pl.DeviceIdType.MESH,
            )

        def do_send(d, h, p, value):
            if h >= 2:
                pl.semaphore_wait(d["credit"], 1)
                rdma(d, h, p, d["to"]).wait_send()
            d["send_buf"][h % 2, p] = value.astype(jnp.bfloat16)
            rdma(d, h, p, d["to"]).start()

        def consume(d, h_recv, p, part):
            rdma(d, h_recv, p, d["frm"]).wait_recv()
            return d["comm_buf"][h_recv % 2, p].astype(jnp.float32) + part

        def credit_back(d, h_recv):
            if h_recv <= N_HOPS - 3:
                pl.semaphore_signal(d["credit"], inc=1,
                                    device_id=(d["frm"],),
                                    device_id_type=pl.DeviceIdType.MESH)

        def epilogue(d, h_recv, blk, p, part):
            acc = consume(d, h_recv, p, part)
            y = acc * scale
            lo = blk * n_blk + p * n_sub
            out_ref[:, lo:lo + n_sub] = (
                y / (1.0 + jnp.exp(-jnp.clip(y, -60.0, 60.0))))
            credit_back(d, h_recv)

        def step_parts(s, ring):
            out = []
            for p in range(SUB):
                row = []
                for d in dirs:
                    c = (my - d["sign"] * (1 + s)) % N_DEV
                    row.append(gemm(c, d["blk0"] + ring, p))
                out.append(row)
            return out

        for t in range(7):
            parts = step_parts(t, 0)
            for p in range(SUB):
                for di, d in enumerate(dirs):
                    part = parts[p][di]
                    acc = part if t == 0 else consume(d, t - 1, p, part)
                    do_send(d, t, p, acc)
                    if t > 0:
                        credit_back(d, t - 1)

        boot_parts = step_parts(0, 1)
        for p in range(SUB):
            for di, d in enumerate(dirs):
                do_send(d, BOOT, p, boot_parts[p][di])
        epi_parts = step_parts(7, 0)
        for p in range(SUB):
            for di, d in enumerate(dirs):
                epilogue(d, BOOT - 1, d["blk0"], p, epi_parts[p][di])

        for t in range(8, 14):
            parts = step_parts(t - 7, 1)
            for p in range(SUB):
                for di, d in enumerate(dirs):
                    acc = consume(d, t - 1, p, parts[p][di])
                    do_send(d, t, p, acc)
                    credit_back(d, t - 1)

        epi_parts = step_parts(7, 1)
        for p in range(SUB):
            for di, d in enumerate(dirs):
                epilogue(d, N_HOPS - 1, d["blk0"] + 1, p, epi_parts[p][di])

        for d in dirs:
            for h in (N_HOPS - 2, N_HOPS - 1):
                for p in range(SUB):
                    rdma(d, h, p, d["to"]).wait_send()

    comm_shape = (2, SUB, m_per, n_sub)
    return pl.pallas_call(
        body,
        out_shape=jax.ShapeDtypeStruct((m_per, n), jnp.float32),
        in_specs=[
            pl.BlockSpec(memory_space=pltpu.VMEM),
            pl.BlockSpec(memory_space=pltpu.VMEM),
            pl.BlockSpec(memory_space=pltpu.SMEM),
            pl.BlockSpec(memory_space=pltpu.SMEM),
        ],
        out_specs=pl.BlockSpec(memory_space=pltpu.VMEM),
        scratch_shapes=[
            pltpu.VMEM(comm_shape, jnp.bfloat16),
            pltpu.VMEM(comm_shape, jnp.bfloat16),
            pltpu.SemaphoreType.DMA((2, SUB)),
            pltpu.SemaphoreType.DMA((2, SUB)),
            pltpu.SemaphoreType.REGULAR,
            pltpu.VMEM(comm_shape, jnp.bfloat16),
            pltpu.VMEM(comm_shape, jnp.bfloat16),
            pltpu.SemaphoreType.DMA((2, SUB)),
            pltpu.SemaphoreType.DMA((2, SUB)),
            pltpu.SemaphoreType.REGULAR,
        ],
        compiler_params=pltpu.CompilerParams(
            collective_id=0, vmem_limit_bytes=int(41.5 * 1024 * 1024)),
    )(x8, w8, scale_x, scale_w)
